# speedup vs baseline: 3.0792x; 3.0792x over previous
import functools

import jax
import jax.numpy as jnp
import numpy as np
from jax import lax
from jax.experimental import pallas as pl
from jax.experimental.pallas import tpu as pltpu
from jax.experimental.pallas import tpu_sc as plsc

N = 10000
E = 160000
D_IN = 256
D_HID = 512
D_OUT = 256
ALPHA = 0.1
THETA = 0.5
NUM_LAYERS = 8

NP = 10240
EP = 163840
NTILES = 32
NB = EP // NTILES // 128
KB = 128
SLICE = NP // 16
BN = 512
NCHUNK = 4

_sc_mesh = plsc.VectorSubcoreMesh(core_axis_name="c", subcore_axis_name="s")


@functools.partial(
    pl.kernel,
    mesh=_sc_mesh,
    out_type=jax.ShapeDtypeStruct((2, NP), jnp.float32),
    scratch_types=[
        pltpu.VMEM((NB, KB), jnp.int32),
        pltpu.VMEM((KB,), jnp.float32),
        pltpu.VMEM((SLICE,), jnp.float32),
        pltpu.VMEM_SHARED((NP,), jnp.float32),
    ],
)
def _deg_kernel(col_hbm, out_hbm, colbuf, ones_v, zbuf, acc):
    c = lax.axis_index("c")
    s = lax.axis_index("s")
    wid = s * 2 + c
    for i in range(KB // 16):
        ones_v[pl.ds(i * 16, 16)] = jnp.ones((16,), jnp.float32)

    def _zb(i, carry):
        zbuf[pl.ds(i * 16, 16)] = jnp.zeros((16,), jnp.float32)
        return carry

    lax.fori_loop(0, SLICE // 16, _zb, 0)
    pltpu.sync_copy(zbuf, acc.at[pl.ds(s * SLICE, SLICE)])
    pltpu.sync_copy(col_hbm.at[wid], colbuf)
    plsc.subcore_barrier()

    def _body(j, carry):
        pltpu.sync_copy(ones_v, acc.at[colbuf.at[j]], add=True)
        return carry

    lax.fori_loop(0, NB, _body, 0)
    plsc.subcore_barrier()
    pltpu.sync_copy(acc.at[pl.ds(s * SLICE, SLICE)],
                    out_hbm.at[c, pl.ds(s * SLICE, SLICE)])


@functools.partial(
    pl.kernel,
    mesh=_sc_mesh,
    out_type=jax.ShapeDtypeStruct((2 * NCHUNK, NP, 128), jnp.float32),
    scratch_types=[
        pltpu.VMEM((NB, KB), jnp.int32),
        pltpu.VMEM((NB, KB), jnp.int32),
        pltpu.VMEM((KB, 128), jnp.float32),
        pltpu.VMEM_SHARED((NP, 128), jnp.float32),
    ],
)
def _spmm_kernel(g0, g1, g2, g3, col_hbm, row_hbm, zeros_hbm, out_hbm,
                 colbuf, rowbuf, gbuf, acc):
    c = lax.axis_index("c")
    s = lax.axis_index("s")
    wid = s * 2 + c
    pltpu.sync_copy(col_hbm.at[wid], colbuf)
    pltpu.sync_copy(row_hbm.at[wid], rowbuf)
    planes = (g0, g1, g2, g3)
    for ch in range(NCHUNK):
        pltpu.sync_copy(zeros_hbm.at[pl.ds(s * SLICE, SLICE)],
                        acc.at[pl.ds(s * SLICE, SLICE)])
        plsc.subcore_barrier()

        def _body(j, carry, plane=planes[ch]):
            pltpu.sync_copy(plane.at[colbuf.at[j]], gbuf)
            pltpu.sync_copy(gbuf, acc.at[rowbuf.at[j]], add=True)
            return carry

        lax.fori_loop(0, NB, _body, 0)
        plsc.subcore_barrier()
        pltpu.sync_copy(acc.at[pl.ds(s * SLICE, SLICE)],
                        out_hbm.at[c * NCHUNK + ch, pl.ds(s * SLICE, SLICE)])


def _dinv_body(p_ref, o_ref):
    deg = p_ref[0] + p_ref[1] + 1.0
    o_ref[...] = lax.rsqrt(deg)


def _dinv_call(p):
    return pl.pallas_call(
        _dinv_body,
        out_shape=jax.ShapeDtypeStruct((NP // 128, 128), jnp.float32),
    )(p)


def _input_body(x_ref, w_ref, b_ref, d_ref, *out_refs):
    h = jnp.maximum(
        jnp.dot(x_ref[...], w_ref[...], preferred_element_type=jnp.float32)
        + b_ref[...], 0.0)
    g = out_refs[:NCHUNK]
    x0 = out_refs[NCHUNK:]
    d = d_ref[...]
    for c2 in range(NCHUNK):
        hc = h[:, c2 * 128:(c2 + 1) * 128]
        x0[c2][...] = hc
        g[c2][...] = d * hc


def _input_call(xp, w_in_t, b_in, dinv_b):
    grid = (NP // BN,)
    outs = [jax.ShapeDtypeStruct((NP, 128), jnp.float32)
            for _ in range(2 * NCHUNK)]
    ospec = [pl.BlockSpec((BN, 128), lambda i: (i, 0))
             for _ in range(2 * NCHUNK)]
    return pl.pallas_call(
        _input_body,
        grid=grid,
        in_specs=[
            pl.BlockSpec((BN, D_IN), lambda i: (i, 0)),
            pl.BlockSpec((D_IN, D_HID), lambda i: (0, 0)),
            pl.BlockSpec((1, D_HID), lambda i: (0, 0)),
            pl.BlockSpec((BN, 128), lambda i: (i, 0)),
        ],
        out_specs=ospec,
        out_shape=outs,
    )(xp, w_in_t, b_in, dinv_b)


def _layer_body(beta, s_ref, g_ref, x0_ref, d_ref, sc_ref, w_ref, *out_refs):
    d = d_ref[...]
    scale = sc_ref[...]
    w = w_ref[...]
    hh = []
    acc = jnp.zeros((BN, D_HID), jnp.float32)
    for k in range(NCHUNK):
        ax = d * (s_ref[k] + s_ref[NCHUNK + k] + g_ref[k][...])
        hk = (1.0 - ALPHA) * ax + ALPHA * x0_ref[k][...]
        hh.append(hk)
        acc = acc + jnp.dot(hk, w[k * 128:(k + 1) * 128, :],
                            preferred_element_type=jnp.float32)
    for c2 in range(NCHUNK):
        res = (1.0 - beta) * hh[c2] + beta * acc[:, c2 * 128:(c2 + 1) * 128]
        out_refs[c2][...] = scale * jnp.maximum(res, 0.0)


def _layer_call(beta, s, g_planes, x0_planes, dinv_b, scale_b, w):
    grid = (NP // BN,)
    outs = [jax.ShapeDtypeStruct((NP, 128), jnp.float32)
            for _ in range(NCHUNK)]
    ospec = [pl.BlockSpec((BN, 128), lambda i: (i, 0)) for _ in range(NCHUNK)]
    plane_spec = pl.BlockSpec((BN, 128), lambda i: (i, 0))
    return pl.pallas_call(
        functools.partial(_layer_body, beta),
        grid=grid,
        in_specs=[
            pl.BlockSpec((2 * NCHUNK, BN, 128), lambda i: (0, i, 0)),
            [plane_spec] * NCHUNK,
            [plane_spec] * NCHUNK,
            pl.BlockSpec((BN, 128), lambda i: (i, 0)),
            pl.BlockSpec((BN, 128), lambda i: (i, 0)),
            pl.BlockSpec((D_HID, D_HID), lambda i: (0, 0)),
        ],
        out_specs=ospec,
        out_shape=outs,
    )(s, list(g_planes), list(x0_planes), dinv_b, scale_b, w)


def _out_body(g_ref, w_ref, b_ref, o_ref):
    acc = jnp.zeros((BN, D_OUT), jnp.float32)
    for k in range(NCHUNK):
        acc = acc + jnp.dot(g_ref[k][...], w_ref[k * 128:(k + 1) * 128, :],
                            preferred_element_type=jnp.float32)
    o_ref[...] = acc + b_ref[...]


def _out_call(g_planes, w_out_t, b_out):
    grid = (NP // BN,)
    plane_spec = pl.BlockSpec((BN, 128), lambda i: (i, 0))
    return pl.pallas_call(
        _out_body,
        grid=grid,
        in_specs=[
            [plane_spec] * NCHUNK,
            pl.BlockSpec((D_HID, D_OUT), lambda i: (0, 0)),
            pl.BlockSpec((1, D_OUT), lambda i: (0, 0)),
        ],
        out_specs=pl.BlockSpec((BN, D_OUT), lambda i: (i, 0)),
        out_shape=jax.ShapeDtypeStruct((NP, D_OUT), jnp.float32),
    )(list(g_planes), w_out_t, b_out)


def kernel(x, edge_index, W_in, b_in, conv_ws, W_out, b_out):
    row = jnp.full((EP,), NP - 1, jnp.int32).at[:E].set(edge_index[0])
    col = jnp.full((EP,), NP - 1, jnp.int32).at[:E].set(edge_index[1])
    row = row.reshape(NTILES, NB, KB)
    col = col.reshape(NTILES, NB, KB)
    xp = jnp.zeros((NP, D_IN), jnp.float32).at[:N].set(x)
    zeros2d = jnp.zeros((NP, 128), jnp.float32)
    ones_b = jnp.ones((NP, 128), jnp.float32)

    p = _deg_kernel(col)
    dinv = _dinv_call(p.reshape(2, NP // 128, 128))
    dinv_b = jnp.broadcast_to(dinv.reshape(NP, 1), (NP, 128))

    planes = _input_call(xp, W_in.T, b_in.reshape(1, D_HID), dinv_b)
    g_planes = planes[:NCHUNK]
    x0_planes = planes[NCHUNK:]

    for l in range(NUM_LAYERS):
        beta = float(np.log(THETA / (l + 1) + 1.0))
        s = _spmm_kernel(*g_planes, col, row, zeros2d)
        scale_b = dinv_b if l < NUM_LAYERS - 1 else ones_b
        g_planes = _layer_call(beta, s, g_planes, x0_planes,
                               dinv_b, scale_b, conv_ws[l])

    out = _out_call(g_planes, W_out.T, b_out.reshape(1, D_OUT))
    return out[:N]
